# Initial kernel scaffold; baseline (speedup 1.0000x reference)
import jax
import jax.numpy as jnp
from jax.experimental import pallas as pl


def kernel(B, M):
    raise NotImplementedError("write your pallas kernel here")


# speedup vs baseline: 1208.3942x; 1208.3942x over previous
import jax
import jax.numpy as jnp
from jax.experimental import pallas as pl
from jax.experimental.pallas import tpu as pltpu

_K = 16
_N = 100000
_NS = 5000

def _gumbel_noise():
    return jax.random.gumbel(jax.random.key(42), (_K, _N), dtype=jnp.float32)


def _select_kernel(b_ref, m_ref, g_ref, mout_ref, logp_ref, bs_ref, key_ref):
    b = b_ref[...]
    row_max = jnp.max(b, axis=1, keepdims=True)
    e = jnp.exp(b - row_max)
    s = jnp.sum(e, axis=1, keepdims=True)
    bs = e / s
    bs_ref[...] = bs
    logits = jnp.log(bs + 1e-30) + g_ref[...]

    u = jax.lax.bitcast_convert_type(logits, jnp.uint32)
    neg = u >= jnp.uint32(0x80000000)
    key = u ^ jnp.where(neg, jnp.uint32(0xFFFFFFFF), jnp.uint32(0x80000000))
    key_ref[...] = key

    def vbody(_, carry):
        lo, hi = carry
        mid = lo + ((hi - lo + jnp.uint32(1)) >> jnp.uint32(1))
        cnt = jnp.sum((key_ref[...] >= mid).astype(jnp.int32), axis=1,
                      keepdims=True)
        ge = cnt >= _NS
        return jnp.where(ge, mid, lo), jnp.where(ge, hi, mid - jnp.uint32(1))

    lo0 = jnp.zeros((_K, 1), jnp.uint32)
    hi0 = jnp.full((_K, 1), jnp.uint32(0xFFFFFFFE))
    thr, _ = jax.lax.fori_loop(0, 32, vbody, (lo0, hi0))

    n_gt = jnp.sum((key_ref[...] > thr).astype(jnp.int32), axis=1,
                   keepdims=True)
    extra = _NS - n_gt
    idx = jax.lax.broadcasted_iota(jnp.int32, (_K, _N), 1)

    def ibody(_, carry):
        lo, hi = carry
        mid = (lo + hi) >> 1
        c = jnp.sum(((key_ref[...] == thr) & (idx <= mid)).astype(jnp.int32),
                    axis=1, keepdims=True)
        ge = c >= extra
        return jnp.where(ge, lo, mid + 1), jnp.where(ge, mid, hi)

    ilo0 = jnp.zeros((_K, 1), jnp.int32)
    ihi0 = jnp.full((_K, 1), _N - 1, jnp.int32)
    istar, _ = jax.lax.fori_loop(0, 17, ibody, (ilo0, ihi0))

    keyv = key_ref[...]
    sel = (keyv > thr) | ((keyv == thr) & (idx <= istar))
    cnt_col = jnp.sum(sel.astype(jnp.float32), axis=0, keepdims=True)

    row = jax.lax.broadcasted_iota(jnp.int32, (_K, _N), 0)
    keep = jnp.where((row == 0) | (cnt_col == 0.0), 1.0, 0.0)
    mout_ref[...] = m_ref[...] * keep

    bsv = bs_ref[...]
    colsum = jnp.sum(bsv, axis=0, keepdims=True)
    contrib = jnp.sum(colsum * cnt_col, keepdims=True)
    m2 = jnp.max(bsv, axis=1, keepdims=True)
    lse2 = jnp.log(jnp.sum(jnp.exp(bsv - m2), axis=1, keepdims=True)) + m2
    logp_ref[...] = contrib - jnp.float32(_K) * jnp.sum(lse2, keepdims=True)


def _pallas_call(interp):
    return pl.pallas_call(
        _select_kernel,
        out_shape=[
            jax.ShapeDtypeStruct((_K, _N), jnp.float32),
            jax.ShapeDtypeStruct((1, 1), jnp.float32),
        ],
        scratch_shapes=[
            pltpu.VMEM((_K, _N), jnp.float32),
            pltpu.VMEM((_K, _N), jnp.uint32),
        ],
        interpret=interp,
    )


def kernel(B, M):
    mout, logp = _pallas_call(False)(B, M, _gumbel_noise())
    return mout, logp[0, 0]
